# Initial kernel scaffold; baseline (speedup 1.0000x reference)
import jax
import jax.numpy as jnp
from jax.experimental import pallas as pl


def kernel(x, ei, ea, Wq1, bq1, Wk1, bk1, Wv1, bv1, We1, Ws1, bs1, Wq2, bq2, Wk2, bk2, Wv2, bv2, We2, Ws2, bs2):
    raise NotImplementedError("write your pallas kernel here")


# speedup vs baseline: 9.2352x; 9.2352x over previous
import functools
import math

import jax
import jax.numpy as jnp
from jax import lax
from jax.experimental import pallas as pl
from jax.experimental.pallas import tpu as pltpu
from jax.experimental.pallas import tpu_sc as plsc


def _mm_body(x_ref, w_ref, b_ref, o_ref):
    o_ref[...] = (
        jnp.dot(x_ref[...], w_ref[...], preferred_element_type=jnp.float32)
        + b_ref[...]
    )


def _matmul(x, w, b, bm):
    m, kdim = x.shape
    _, n = w.shape
    return pl.pallas_call(
        _mm_body,
        grid=(m // bm,),
        in_specs=[
            pl.BlockSpec((bm, kdim), lambda i: (i, 0)),
            pl.BlockSpec((kdim, n), lambda i: (0, 0)),
            pl.BlockSpec((1, n), lambda i: (0, 0)),
        ],
        out_specs=pl.BlockSpec((bm, n), lambda i: (i, 0)),
        out_shape=jax.ShapeDtypeStruct((m, n), jnp.float32),
    )(x, w, b.reshape(1, -1))


def _combine_body(num_ref, den_ref, skip_ref, h_ref, o_ref, *, leaky):
    num = num_ref[0] + num_ref[1]
    den = den_ref[0] + den_ref[1]
    den_full = jnp.dot(den, h_ref[...], preferred_element_type=jnp.float32)
    out = num / (den_full + 1e-16) + skip_ref[...]
    if leaky:
        out = jnp.where(out >= 0, out, 0.01 * out)
    o_ref[...] = out


def _combine(num, den, skip, hmat, leaky, bm):
    n = skip.shape[0]
    return pl.pallas_call(
        functools.partial(_combine_body, leaky=leaky),
        grid=(n // bm,),
        in_specs=[
            pl.BlockSpec((2, bm, 128), lambda i: (0, i, 0)),
            pl.BlockSpec((2, bm, 16), lambda i: (0, i, 0)),
            pl.BlockSpec((bm, 128), lambda i: (i, 0)),
            pl.BlockSpec((16, 128), lambda i: (0, 0)),
        ],
        out_specs=pl.BlockSpec((bm, 128), lambda i: (i, 0)),
        out_shape=jax.ShapeDtypeStruct((n, 128), jnp.float32),
    )(num, den, skip, hmat)


def _edge_pass(q, k, v, e, src, dst, heads):
    n, d = q.shape
    num_e = src.shape[0]
    info = plsc.get_sparse_core_info()
    nc, ns = info.num_cores, info.num_subcores
    nw = nc * ns
    epw = num_e // nw
    blk_e = 16
    nblk = epw // blk_e
    rows_pt = (n // ns) // 16 * 16
    rows_tail = n - ns * rows_pt
    scale = 1.0 / math.sqrt(d // heads)
    cw = d // heads
    mesh = plsc.VectorSubcoreMesh(core_axis_name="c", subcore_axis_name="s")

    @functools.partial(
        pl.kernel,
        out_type=[
            jax.ShapeDtypeStruct((nc, n, 128), jnp.float32),
            jax.ShapeDtypeStruct((nc, n, 16), jnp.float32),
        ],
        mesh=mesh,
        compiler_params=pltpu.CompilerParams(needs_layout_passes=False),
        scratch_types=[
            pltpu.VMEM((blk_e,), jnp.int32),
            pltpu.VMEM((blk_e,), jnp.int32),
            pltpu.VMEM((blk_e, 128), jnp.float32),
            pltpu.VMEM((blk_e, 128), jnp.float32),
            pltpu.VMEM((blk_e, 128), jnp.float32),
            pltpu.VMEM((blk_e, 128), jnp.float32),
            pltpu.VMEM((blk_e, 16), jnp.float32),
            pltpu.VMEM((256,), jnp.float32),
            pltpu.VMEM((2048,), jnp.float32),
            pltpu.VMEM_SHARED((n, 128), jnp.float32),
            pltpu.VMEM_SHARED((n, 16), jnp.float32),
            pltpu.SemaphoreType.DMA,
            pltpu.SemaphoreType.DMA,
            pltpu.SemaphoreType.DMA,
            pltpu.SemaphoreType.DMA,
        ],
    )
    def ker(q_hbm, k_hbm, v_hbm, e_hbm, src_hbm, dst_hbm,
            num_out, den_out, srcidx, dstidx, qbuf, kbuf, vbuf, ebuf,
            wout, wtmp, pbuf, numacc, denacc, s0, s1, s2, s3):
        cid = lax.axis_index("c")
        sid = lax.axis_index("s")
        wid = cid * ns + sid
        r0 = sid * rows_pt
        iota16 = lax.iota(jnp.int32, 16)
        zero16 = jnp.zeros((16,), jnp.float32)
        for j in range(16):
            for t in range(8):
                qbuf[j, pl.ds(16 * t, 16)] = zero16
            wout[j, :] = zero16
        for hh in range(16):
            wtmp[pl.ds(16 * hh, 16)] = zero16
        for i in range(rows_pt // 16):
            srcidx[...] = r0 + i * 16 + iota16
            pltpu.sync_copy(qbuf, numacc.at[srcidx])
            pltpu.sync_copy(wout, denacc.at[srcidx])

        @pl.when(sid == 0)
        def _zero_tail():
            srcidx[...] = ns * rows_pt + iota16
            pltpu.sync_copy(qbuf, numacc.at[srcidx])
            pltpu.sync_copy(wout, denacc.at[srcidx])
        plsc.subcore_barrier()

        ebase0 = wid * epw
        iota16 = lax.iota(jnp.int32, 16)

        def block_body(bi, carry):
            base = ebase0 + bi * blk_e
            pltpu.sync_copy(src_hbm.at[pl.ds(base, blk_e)], srcidx)
            pltpu.sync_copy(dst_hbm.at[pl.ds(base, blk_e)], dstidx)
            cq = pltpu.async_copy(q_hbm.at[dstidx], qbuf, s0)
            ck = pltpu.async_copy(k_hbm.at[srcidx], kbuf, s1)
            cv = pltpu.async_copy(v_hbm.at[srcidx], vbuf, s2)
            ce = pltpu.async_copy(e_hbm.at[pl.ds(base, blk_e)], ebuf, s3)
            cq.wait()
            ck.wait()
            cv.wait()
            ce.wait()

            for j in range(16):
                for t in range(8):
                    sl = pl.ds(16 * t, 16)
                    pbuf[pl.ds(j * 128 + 16 * t, 16)] = (
                        qbuf[j, sl] * (kbuf[j, sl] + ebuf[j, sl])
                    )
            edge_base = iota16 * 128
            for h in range(heads):
                s = jnp.zeros((16,), jnp.float32)
                for c in range(h * cw, (h + 1) * cw):
                    s = s + plsc.load_gather(pbuf, [edge_base + c])
                wtmp[pl.ds(16 * h, 16)] = jnp.exp(s * scale)
            for j in range(16):
                wout[j, :] = plsc.load_gather(wtmp, [iota16 * 16 + j])
                for h in range(heads):
                    w = plsc.load_gather(
                        wtmp, [jnp.full((16,), 16 * h + j, jnp.int32)]
                    )
                    for t2 in range(cw // 16):
                        sl = pl.ds(h * cw + 16 * t2, 16)
                        vbuf[j, sl] = (vbuf[j, sl] + ebuf[j, sl]) * w

            pltpu.sync_copy(vbuf, numacc.at[dstidx], add=True)
            pltpu.sync_copy(wout, denacc.at[dstidx], add=True)
            return carry

        lax.fori_loop(0, nblk, block_body, 0)
        plsc.subcore_barrier()
        for i in range(rows_pt // 16):
            srcidx[...] = r0 + i * 16 + iota16
            pltpu.sync_copy(numacc.at[srcidx], qbuf)
            pltpu.sync_copy(qbuf, num_out.at[cid, pl.ds(r0 + i * 16, 16)])
            pltpu.sync_copy(denacc.at[srcidx], wout)
            pltpu.sync_copy(wout, den_out.at[cid, pl.ds(r0 + i * 16, 16)])

        @pl.when(sid == 0)
        def _write_tail():
            rt = ns * rows_pt
            srcidx[...] = rt + iota16
            pltpu.sync_copy(numacc.at[srcidx], qbuf)
            pltpu.sync_copy(qbuf, num_out.at[cid, pl.ds(rt, rows_tail)])
            pltpu.sync_copy(denacc.at[srcidx], wout)
            pltpu.sync_copy(wout, den_out.at[cid, pl.ds(rt, rows_tail)])

    return ker(q, k, v, e, src, dst)


def _layer(h, src, dst, ea, Wq, bq, Wk, bk, Wv, bv, We, Ws, bs, heads, leaky):
    n = h.shape[0]
    wcat = jnp.concatenate([Wq, Wk, Wv, Ws], axis=1)
    bcat = jnp.concatenate([bq, bk, bv, bs])
    proj = _matmul(h, wcat, bcat, bm=400)
    q = proj[:, 0:128]
    k = proj[:, 128:256]
    v = proj[:, 256:384]
    skip = proj[:, 384:512]
    e = _matmul(ea, We, jnp.zeros((128,), jnp.float32), bm=2000)
    num, den = _edge_pass(q, k, v, e, src, dst, heads)
    hsel = (jnp.arange(16)[:, None] == (jnp.arange(128)[None, :] // (128 // heads)))
    hmat = hsel.astype(jnp.float32)
    return _combine(num, den, skip, hmat, leaky, bm=400)


def kernel(x, ei, ea, Wq1, bq1, Wk1, bk1, Wv1, bv1, We1, Ws1, bs1,
           Wq2, bq2, Wk2, bk2, Wv2, bv2, We2, Ws2, bs2):
    src = ei[0]
    dst = ei[1]
    h = _layer(x, src, dst, ea, Wq1, bq1, Wk1, bk1, Wv1, bv1, We1, Ws1, bs1,
               heads=8, leaky=True)
    return _layer(h, src, dst, ea, Wq2, bq2, Wk2, bk2, Wv2, bv2, We2, Ws2, bs2,
                  heads=1, leaky=False)


# speedup vs baseline: 9.8185x; 1.0632x over previous
import functools
import math

import jax
import jax.numpy as jnp
from jax import lax
from jax.experimental import pallas as pl
from jax.experimental.pallas import tpu as pltpu
from jax.experimental.pallas import tpu_sc as plsc


def _mm_body(x_ref, w_ref, b_ref, o_ref):
    o_ref[...] = (
        jnp.dot(x_ref[...], w_ref[...], preferred_element_type=jnp.float32)
        + b_ref[...]
    )


def _matmul(x, w, b, bm):
    m, kdim = x.shape
    _, n = w.shape
    return pl.pallas_call(
        _mm_body,
        grid=(m // bm,),
        in_specs=[
            pl.BlockSpec((bm, kdim), lambda i: (i, 0)),
            pl.BlockSpec((kdim, n), lambda i: (0, 0)),
            pl.BlockSpec((1, n), lambda i: (0, 0)),
        ],
        out_specs=pl.BlockSpec((bm, n), lambda i: (i, 0)),
        out_shape=jax.ShapeDtypeStruct((m, n), jnp.float32),
    )(x, w, b.reshape(1, -1))


def _combine_body(num_ref, den_ref, skip_ref, h_ref, o_ref, *, leaky):
    num = num_ref[0] + num_ref[1]
    den = den_ref[0] + den_ref[1]
    den_full = jnp.dot(den, h_ref[...], preferred_element_type=jnp.float32)
    out = num / (den_full + 1e-16) + skip_ref[...]
    if leaky:
        out = jnp.where(out >= 0, out, 0.01 * out)
    o_ref[...] = out


def _combine(num, den, skip, hmat, leaky, bm):
    n = skip.shape[0]
    return pl.pallas_call(
        functools.partial(_combine_body, leaky=leaky),
        grid=(n // bm,),
        in_specs=[
            pl.BlockSpec((2, bm, 128), lambda i: (0, i, 0)),
            pl.BlockSpec((2, bm, 16), lambda i: (0, i, 0)),
            pl.BlockSpec((bm, 128), lambda i: (i, 0)),
            pl.BlockSpec((16, 128), lambda i: (0, 0)),
        ],
        out_specs=pl.BlockSpec((bm, 128), lambda i: (i, 0)),
        out_shape=jax.ShapeDtypeStruct((n, 128), jnp.float32),
    )(num, den, skip, hmat)


def _edge_pass(q, k, v, e, src, dst, heads):
    n, d = q.shape
    num_e = src.shape[0]
    info = plsc.get_sparse_core_info()
    nc, ns = info.num_cores, info.num_subcores
    nw = nc * ns
    epw = num_e // nw
    blk_e = 16
    nblk = epw // blk_e
    rows_pt = (n // ns) // 16 * 16
    rows_tail = n - ns * rows_pt
    scale = 1.0 / math.sqrt(d // heads)
    cw = d // heads
    mesh = plsc.VectorSubcoreMesh(core_axis_name="c", subcore_axis_name="s")

    @functools.partial(
        pl.kernel,
        out_type=[
            jax.ShapeDtypeStruct((nc, n, 128), jnp.float32),
            jax.ShapeDtypeStruct((nc, n, 16), jnp.float32),
        ],
        mesh=mesh,
        compiler_params=pltpu.CompilerParams(needs_layout_passes=False),
        scratch_types=[
            pltpu.VMEM((blk_e,), jnp.int32),
            pltpu.VMEM((blk_e,), jnp.int32),
            pltpu.VMEM((blk_e,), jnp.int32),
            pltpu.VMEM((blk_e,), jnp.int32),
            pltpu.VMEM((blk_e, 128), jnp.float32),
            pltpu.VMEM((blk_e, 128), jnp.float32),
            pltpu.VMEM((blk_e, 128), jnp.float32),
            pltpu.VMEM((blk_e, 128), jnp.float32),
            pltpu.VMEM((blk_e, 128), jnp.float32),
            pltpu.VMEM((blk_e, 128), jnp.float32),
            pltpu.VMEM((blk_e, 128), jnp.float32),
            pltpu.VMEM((blk_e, 128), jnp.float32),
            pltpu.VMEM((blk_e, 16), jnp.float32),
            pltpu.VMEM((256,), jnp.float32),
            pltpu.VMEM((2048,), jnp.float32),
            pltpu.VMEM_SHARED((n, 128), jnp.float32),
            pltpu.VMEM_SHARED((n, 16), jnp.float32),
            pltpu.SemaphoreType.DMA,
            pltpu.SemaphoreType.DMA,
            pltpu.SemaphoreType.DMA,
            pltpu.SemaphoreType.DMA,
            pltpu.SemaphoreType.DMA,
            pltpu.SemaphoreType.DMA,
            pltpu.SemaphoreType.DMA,
            pltpu.SemaphoreType.DMA,
        ],
    )
    def ker(q_hbm, k_hbm, v_hbm, e_hbm, src_hbm, dst_hbm,
            num_out, den_out, srcidx, dstidx, srcidx2, dstidx2,
            qbuf, kbuf, vbuf, ebuf, qbuf2, kbuf2, vbuf2, ebuf2,
            wout, wtmp, pbuf, numacc, denacc,
            s0, s1, s2, s3, s4, s5, s6, s7):
        cid = lax.axis_index("c")
        sid = lax.axis_index("s")
        wid = cid * ns + sid
        r0 = sid * rows_pt
        iota16 = lax.iota(jnp.int32, 16)
        zero16 = jnp.zeros((16,), jnp.float32)
        for j in range(16):
            for t in range(8):
                qbuf[j, pl.ds(16 * t, 16)] = zero16
            wout[j, :] = zero16
        for hh in range(16):
            wtmp[pl.ds(16 * hh, 16)] = zero16
        for i in range(rows_pt // 16):
            srcidx[...] = r0 + i * 16 + iota16
            pltpu.sync_copy(qbuf, numacc.at[srcidx])
            pltpu.sync_copy(wout, denacc.at[srcidx])

        @pl.when(sid == 0)
        def _zero_tail():
            srcidx[...] = ns * rows_pt + iota16
            pltpu.sync_copy(qbuf, numacc.at[srcidx])
            pltpu.sync_copy(wout, denacc.at[srcidx])
        plsc.subcore_barrier()

        ebase0 = wid * epw
        iota16 = lax.iota(jnp.int32, 16)
        bufA = (srcidx, dstidx, qbuf, kbuf, vbuf, ebuf, s0, s1, s2, s3)
        bufB = (srcidx2, dstidx2, qbuf2, kbuf2, vbuf2, ebuf2, s4, s5, s6, s7)

        def load_block(bi, buf):
            si, di, qb, kb, vb, eb, t0, t1, t2, t3 = buf
            base = ebase0 + bi * blk_e
            pltpu.sync_copy(src_hbm.at[pl.ds(base, blk_e)], si)
            pltpu.sync_copy(dst_hbm.at[pl.ds(base, blk_e)], di)
            pltpu.async_copy(q_hbm.at[di], qb, t0)
            pltpu.async_copy(k_hbm.at[si], kb, t1)
            pltpu.async_copy(v_hbm.at[si], vb, t2)
            pltpu.async_copy(e_hbm.at[pl.ds(base, blk_e)], eb, t3)

        def compute_block(bi, buf):
            si, di, qb, kb, vb, eb, t0, t1, t2, t3 = buf
            base = ebase0 + bi * blk_e
            pltpu.make_async_copy(q_hbm.at[di], qb, t0).wait()
            pltpu.make_async_copy(k_hbm.at[si], kb, t1).wait()
            pltpu.make_async_copy(v_hbm.at[si], vb, t2).wait()
            pltpu.make_async_copy(e_hbm.at[pl.ds(base, blk_e)], eb, t3).wait()

            for j in range(16):
                for t in range(8):
                    sl = pl.ds(16 * t, 16)
                    pbuf[pl.ds(j * 128 + 16 * t, 16)] = (
                        qb[j, sl] * (kb[j, sl] + eb[j, sl])
                    )
            edge_base = iota16 * 128
            for h in range(heads):
                s = jnp.zeros((16,), jnp.float32)
                for c in range(h * cw, (h + 1) * cw):
                    s = s + plsc.load_gather(pbuf, [edge_base + c])
                wtmp[pl.ds(16 * h, 16)] = jnp.exp(s * scale)
            for j in range(16):
                wout[j, :] = plsc.load_gather(wtmp, [iota16 * 16 + j])
                for h in range(heads):
                    w = plsc.load_gather(
                        wtmp, [jnp.full((16,), 16 * h + j, jnp.int32)]
                    )
                    for t2 in range(cw // 16):
                        sl = pl.ds(h * cw + 16 * t2, 16)
                        vb[j, sl] = (vb[j, sl] + eb[j, sl]) * w

            pltpu.sync_copy(vb, numacc.at[di], add=True)
            pltpu.sync_copy(wout, denacc.at[di], add=True)

        load_block(0, bufA)

        def pair_body(g, carry):
            i = g * 2
            load_block(i + 1, bufB)
            compute_block(i, bufA)
            load_block(i + 2, bufA)
            compute_block(i + 1, bufB)
            return carry

        lax.fori_loop(0, (nblk - 1) // 2, pair_body, 0)
        compute_block(nblk - 1, bufA)
        plsc.subcore_barrier()
        for i in range(rows_pt // 16):
            srcidx[...] = r0 + i * 16 + iota16
            pltpu.sync_copy(numacc.at[srcidx], qbuf)
            pltpu.sync_copy(qbuf, num_out.at[cid, pl.ds(r0 + i * 16, 16)])
            pltpu.sync_copy(denacc.at[srcidx], wout)
            pltpu.sync_copy(wout, den_out.at[cid, pl.ds(r0 + i * 16, 16)])

        @pl.when(sid == 0)
        def _write_tail():
            rt = ns * rows_pt
            srcidx[...] = rt + iota16
            pltpu.sync_copy(numacc.at[srcidx], qbuf)
            pltpu.sync_copy(qbuf, num_out.at[cid, pl.ds(rt, rows_tail)])
            pltpu.sync_copy(denacc.at[srcidx], wout)
            pltpu.sync_copy(wout, den_out.at[cid, pl.ds(rt, rows_tail)])

    return ker(q, k, v, e, src, dst)


def _layer(h, src, dst, ea, Wq, bq, Wk, bk, Wv, bv, We, Ws, bs, heads, leaky):
    n = h.shape[0]
    wcat = jnp.concatenate([Wq, Wk, Wv, Ws], axis=1)
    bcat = jnp.concatenate([bq, bk, bv, bs])
    proj = _matmul(h, wcat, bcat, bm=400)
    q = proj[:, 0:128]
    k = proj[:, 128:256]
    v = proj[:, 256:384]
    skip = proj[:, 384:512]
    e = _matmul(ea, We, jnp.zeros((128,), jnp.float32), bm=2000)
    num, den = _edge_pass(q, k, v, e, src, dst, heads)
    hsel = (jnp.arange(16)[:, None] == (jnp.arange(128)[None, :] // (128 // heads)))
    hmat = hsel.astype(jnp.float32)
    return _combine(num, den, skip, hmat, leaky, bm=400)


def kernel(x, ei, ea, Wq1, bq1, Wk1, bk1, Wv1, bv1, We1, Ws1, bs1,
           Wq2, bq2, Wk2, bk2, Wv2, bv2, We2, Ws2, bs2):
    src = ei[0]
    dst = ei[1]
    h = _layer(x, src, dst, ea, Wq1, bq1, Wk1, bk1, Wv1, bv1, We1, Ws1, bs1,
               heads=8, leaky=True)
    return _layer(h, src, dst, ea, Wq2, bq2, Wk2, bk2, Wv2, bv2, We2, Ws2, bs2,
                  heads=1, leaky=False)
